# Initial kernel scaffold; baseline (speedup 1.0000x reference)
import jax
import jax.numpy as jnp
from jax.experimental import pallas as pl


def kernel(p1, p3, p5, idx_i, idx_j, diff, basis, pp1_W, pp1_b, pi1_W1, pi1_b1, pi1_W2, ii1_W, ii1_b, pp3_W, pp3_b, ii3_W, ii3_b, pp5_W, pp5_b):
    raise NotImplementedError("write your pallas kernel here")


# speedup vs baseline: 15.9329x; 15.9329x over previous
import functools

import jax
import jax.numpy as jnp
from jax import lax
from jax.experimental import pallas as pl
from jax.experimental.pallas import tpu as pltpu
from jax.experimental.pallas import tpu_sc as plsc

N = 10000
E = 160000
C = 16
NB = 16
CHUNK = 128
NCHUNKS = E // CHUNK
NWORKERS = 32
STEPS = -(-NCHUNKS // NWORKERS)

_mesh = plsc.VectorSubcoreMesh(core_axis_name="c", subcore_axis_name="s")
_sc_params = pltpu.CompilerParams(use_tc_tiling_on_sc=False)


def _tc1_body(p1, p3r, Wp1, bp1, WA, cA, WB, Wp3, bp3,
              p1n, An, Bn, p3n):
    x = p1[...]
    f32 = jnp.float32
    p1n[...] = jnp.dot(x, Wp1[...], preferred_element_type=f32) + bp1[...]
    An[...] = jnp.dot(x, WA[...], preferred_element_type=f32) + cA[...]
    Bn[...] = jnp.dot(x, WB[...], preferred_element_type=f32)
    p3n[...] = jnp.dot(p3r[...], Wp3[...], preferred_element_type=f32) + bp3[...]


@functools.partial(
    pl.kernel,
    out_type=(jax.ShapeDtypeStruct((E, C), jnp.float32),
              jax.ShapeDtypeStruct((E, C), jnp.float32),
              jax.ShapeDtypeStruct((E, 3 * C), jnp.float32)),
    mesh=_mesh,
    scratch_types=[pltpu.VMEM((CHUNK,), jnp.int32),
                   pltpu.VMEM((CHUNK,), jnp.int32),
                   pltpu.VMEM((CHUNK, C), jnp.float32),
                   pltpu.VMEM((CHUNK, C), jnp.float32),
                   pltpu.VMEM((CHUNK, 3 * C), jnp.float32),
                   pltpu.SemaphoreType.DMA,
                   pltpu.SemaphoreType.DMA,
                   pltpu.SemaphoreType.DMA],
    compiler_params=_sc_params,
)
def _sc1_gather(An, Bn, p3n, ii2, ij2, gA, gB, g3,
                ivi, ivj, bA, bB, b3, s1, s2, s3):
    wid = lax.axis_index("s") * 2 + lax.axis_index("c")

    def step(k, carry):
        c = wid + NWORKERS * k

        @pl.when(c < NCHUNKS)
        def _():
            pltpu.sync_copy(ii2.at[c], ivi)
            pltpu.sync_copy(ij2.at[c], ivj)
            d1 = pltpu.async_copy(An.at[ivi], bA, s1)
            d2 = pltpu.async_copy(Bn.at[ivj], bB, s2)
            d3 = pltpu.async_copy(p3n.at[ivj], b3, s3)
            d1.wait()
            d2.wait()
            d3.wait()
            pltpu.sync_copy(bA, gA.at[pl.ds(c * CHUNK, CHUNK)])
            pltpu.sync_copy(bB, gB.at[pl.ds(c * CHUNK, CHUNK)])
            pltpu.sync_copy(b3, g3.at[pl.ds(c * CHUNK, CHUNK)])

        return carry

    lax.fori_loop(0, STEPS, step, 0)


def _tc2_body(gA, gB, basis, g3, diff, W2, T, Rs, W14, b14, Wbd, b3t,
              o1, o3):
    f32 = jnp.float32
    h = gA[...] + gB[...]
    ht = jnp.dot(h, W2[...], preferred_element_type=f32)
    br = jnp.dot(basis[...], T[...], preferred_element_type=f32)
    i1 = jnp.dot(ht * br, Rs[...], preferred_element_type=f32)
    i14 = jnp.dot(i1, W14[...], preferred_element_type=f32) + b14[...]
    i1_2 = i14[:, C:2 * C]
    i1_3 = i14[:, 2 * C:3 * C]
    y = g3[...] * jnp.concatenate([i1_2, i1_2, i1_2], axis=1)
    d = diff[...]
    sd = jnp.concatenate([d[:, 0:1] * i1_3, d[:, 1:2] * i1_3,
                          d[:, 2:3] * i1_3], axis=1)
    o1[...] = i14[:, :C]
    o3[...] = jnp.dot(y, Wbd[...], preferred_element_type=f32) + b3t[...] + sd


@functools.partial(
    pl.kernel,
    out_type=(jax.ShapeDtypeStruct((2, N, C), jnp.float32),
              jax.ShapeDtypeStruct((2, N, 3 * C), jnp.float32)),
    mesh=_mesh,
    scratch_types=[pltpu.VMEM((CHUNK,), jnp.int32),
                   pltpu.VMEM((CHUNK, C), jnp.float32),
                   pltpu.VMEM((CHUNK, 3 * C), jnp.float32),
                   pltpu.VMEM_SHARED((N, C), jnp.float32),
                   pltpu.VMEM_SHARED((N, 3 * C), jnp.float32),
                   pltpu.SemaphoreType.DMA,
                   pltpu.SemaphoreType.DMA],
    compiler_params=_sc_params,
)
def _sc2_scatter(ii2, e1, e3, z16, z48, o1, o3,
                 iv, b1, b3, acc1, acc3, s1, s2):
    cid = lax.axis_index("c")
    sid = lax.axis_index("s")
    wid = sid * 2 + cid

    @pl.when(sid == 0)
    def _():
        pltpu.sync_copy(z16, acc1)
        pltpu.sync_copy(z48, acc3)

    plsc.subcore_barrier()

    def step(k, carry):
        c = wid + NWORKERS * k

        @pl.when(c < NCHUNKS)
        def _():
            pltpu.sync_copy(ii2.at[c], iv)
            d1 = pltpu.async_copy(e1.at[pl.ds(c * CHUNK, CHUNK)], b1, s1)
            d2 = pltpu.async_copy(e3.at[pl.ds(c * CHUNK, CHUNK)], b3, s2)
            d1.wait()
            d2.wait()
            pltpu.sync_copy(b1, acc1.at[iv], add=True)
            pltpu.sync_copy(b3, acc3.at[iv], add=True)

        return carry

    lax.fori_loop(0, STEPS, step, 0)
    plsc.subcore_barrier()

    @pl.when(sid == 0)
    def _():
        pltpu.sync_copy(acc1, o1.at[cid])
        pltpu.sync_copy(acc3, o3.at[cid])


def _tc3_body(p1n, p3n, o1, o3, np1, np3):
    p1a = p1n[...] + o1[0] + o1[1]
    p3a = p3n[...] + o3[0] + o3[1]
    np1[...] = p1a + p1a * p1a
    np3[...] = p3a * jnp.concatenate([p1a, p1a, p1a], axis=1)


def _full_spec(shape):
    return pl.BlockSpec(shape, lambda *_: tuple(0 for _ in shape))


def kernel(p1, p3, p5, idx_i, idx_j, diff, basis,
           pp1_W, pp1_b, pi1_W1, pi1_b1, pi1_W2, ii1_W, ii1_b,
           pp3_W, pp3_b, ii3_W, ii3_b, pp5_W, pp5_b):
    f32 = jnp.float32
    del p5, pp5_W, pp5_b

    W1a, W1b = pi1_W1[:C], pi1_W1[C:]
    c_h = (pp1_b @ W1a + pp1_b @ W1b + pi1_b1)[None, :]
    WA = pp1_W @ W1a
    WB = pp1_W @ W1b
    T = jnp.kron(jnp.ones((1, C), f32), jnp.eye(NB, dtype=f32))
    Rs = jnp.kron(jnp.eye(C, dtype=f32), jnp.ones((NB, 1), f32))
    W14 = ii1_W[:, :3 * C]
    b14 = ii1_b[None, :3 * C]
    Wbd = jnp.kron(jnp.eye(3, dtype=f32), ii3_W)
    b3t = jnp.tile(ii3_b, 3)[None, :]

    p3r = p3.reshape(3 * N, C)
    ii2 = idx_i.reshape(NCHUNKS, CHUNK)
    ij2 = idx_j.reshape(NCHUNKS, CHUNK)

    p1n, An, Bn, p3nr = pl.pallas_call(
        _tc1_body,
        out_shape=(jax.ShapeDtypeStruct((N, C), f32),
                   jax.ShapeDtypeStruct((N, C), f32),
                   jax.ShapeDtypeStruct((N, C), f32),
                   jax.ShapeDtypeStruct((3 * N, C), f32)),
    )(p1, p3r, pp1_W, pp1_b[None, :], WA, c_h, WB, pp3_W, pp3_b[None, :])
    p3n = p3nr.reshape(N, 3 * C)

    gA, gB, g3 = _sc1_gather(An, Bn, p3n, ii2, ij2)

    TE = 2000
    grid = (E // TE,)
    espec = lambda w: pl.BlockSpec((TE, w), lambda i: (i, 0))
    o1, o3 = pl.pallas_call(
        _tc2_body,
        grid=grid,
        in_specs=[espec(C), espec(C), espec(NB), espec(3 * C), espec(3),
                  _full_spec((C, C * NB)), _full_spec((NB, C * NB)),
                  _full_spec((C * NB, C)), _full_spec((C, 3 * C)),
                  _full_spec((1, 3 * C)), _full_spec((3 * C, 3 * C)),
                  _full_spec((1, 3 * C))],
        out_specs=[espec(C), espec(3 * C)],
        out_shape=(jax.ShapeDtypeStruct((E, C), f32),
                   jax.ShapeDtypeStruct((E, 3 * C), f32)),
    )(gA, gB, basis, g3, diff, pi1_W2, T, Rs, W14, b14, Wbd, b3t)

    z16 = jnp.zeros((N, C), f32)
    z48 = jnp.zeros((N, 3 * C), f32)
    a1, a3 = _sc2_scatter(ii2, o1, o3, z16, z48)

    np1, np3 = pl.pallas_call(
        _tc3_body,
        out_shape=(jax.ShapeDtypeStruct((N, C), f32),
                   jax.ShapeDtypeStruct((N, 3 * C), f32)),
    )(p1n, p3n, a1, a3)
    return (np1, np3.reshape(N, 3, C))


# speedup vs baseline: 22.6581x; 1.4221x over previous
import functools

import jax
import jax.numpy as jnp
from jax import lax
from jax.experimental import pallas as pl
from jax.experimental.pallas import tpu as pltpu
from jax.experimental.pallas import tpu_sc as plsc

N = 10000
E = 160000
C = 16
NB = 16
CHUNK = 128
NCHUNKS = E // CHUNK
NWORKERS = 32
STEPS = -(-NCHUNKS // NWORKERS)
NP8 = N // 8
EP8 = E // 8

_mesh = plsc.VectorSubcoreMesh(core_axis_name="c", subcore_axis_name="s")
_sc_params = pltpu.CompilerParams(use_tc_tiling_on_sc=False)


def _tc1_body(p1, p3r, Wp1, bp1, WA, cA, WB, Wp3, bp3,
              p1n, An, Bn, p3n):
    x = p1[...]
    f32 = jnp.float32
    p1n[...] = jnp.dot(x, Wp1[...], preferred_element_type=f32) + bp1[...]
    An[...] = jnp.dot(x, WA[...], preferred_element_type=f32) + cA[...]
    Bn[...] = jnp.dot(x, WB[...], preferred_element_type=f32)
    p3n[...] = jnp.dot(p3r[...], Wp3[...], preferred_element_type=f32) + bp3[...]


@functools.partial(
    pl.kernel,
    out_type=(jax.ShapeDtypeStruct((E, C), jnp.float32),
              jax.ShapeDtypeStruct((E, C), jnp.float32),
              jax.ShapeDtypeStruct((E, 3 * C), jnp.float32)),
    mesh=_mesh,
    scratch_types=[pltpu.VMEM((CHUNK,), jnp.int32),
                   pltpu.VMEM((CHUNK,), jnp.int32),
                   pltpu.VMEM((CHUNK, C), jnp.float32),
                   pltpu.VMEM((CHUNK, C), jnp.float32),
                   pltpu.VMEM((CHUNK, 3 * C), jnp.float32),
                   pltpu.SemaphoreType.DMA,
                   pltpu.SemaphoreType.DMA,
                   pltpu.SemaphoreType.DMA],
    compiler_params=_sc_params,
)
def _sc1_gather(An, Bn, p3n, ii2, ij2, gA, gB, g3,
                ivi, ivj, bA, bB, b3, s1, s2, s3):
    wid = lax.axis_index("s") * 2 + lax.axis_index("c")

    def step(k, carry):
        c = wid + NWORKERS * k

        @pl.when(c < NCHUNKS)
        def _():
            pltpu.sync_copy(ii2.at[c], ivi)
            pltpu.sync_copy(ij2.at[c], ivj)
            d1 = pltpu.async_copy(An.at[ivi], bA, s1)
            d2 = pltpu.async_copy(Bn.at[ivj], bB, s2)
            d3 = pltpu.async_copy(p3n.at[ivj], b3, s3)
            d1.wait()
            d2.wait()
            d3.wait()
            pltpu.sync_copy(bA, gA.at[pl.ds(c * CHUNK, CHUNK)])
            pltpu.sync_copy(bB, gB.at[pl.ds(c * CHUNK, CHUNK)])
            pltpu.sync_copy(b3, g3.at[pl.ds(c * CHUNK, CHUNK)])

        return carry

    lax.fori_loop(0, STEPS, step, 0)


def _tc2_body(gA, gB, basis, g3, diffp, W2K, TK, RsK, O1K, T2K, T3K,
              WbdK, Md, b1r, b2r, b3r, bb3r, o1, o3):
    f32 = jnp.float32
    h = gA[...] + gB[...]
    ht = jnp.dot(h, W2K[...], preferred_element_type=f32)
    br = jnp.dot(basis[...], TK[...], preferred_element_type=f32)
    i1 = jnp.dot(ht * br, RsK[...], preferred_element_type=f32)
    o1[...] = jnp.dot(i1, O1K[...], preferred_element_type=f32) + b1r[...]
    i2t = jnp.dot(i1, T2K[...], preferred_element_type=f32) + b2r[...]
    i3t = jnp.dot(i1, T3K[...], preferred_element_type=f32) + b3r[...]
    y = g3[...] * i2t
    dexp = jnp.dot(diffp[...], Md[...], preferred_element_type=f32)
    o3[...] = (jnp.dot(y, WbdK[...], preferred_element_type=f32)
               + dexp * i3t + bb3r[...])


@functools.partial(
    pl.kernel,
    out_type=(jax.ShapeDtypeStruct((2, N, C), jnp.float32),
              jax.ShapeDtypeStruct((2, N, 3 * C), jnp.float32)),
    mesh=_mesh,
    scratch_types=[pltpu.VMEM((CHUNK,), jnp.int32),
                   pltpu.VMEM((CHUNK, C), jnp.float32),
                   pltpu.VMEM((CHUNK, 3 * C), jnp.float32),
                   pltpu.VMEM_SHARED((N, C), jnp.float32),
                   pltpu.VMEM_SHARED((N, 3 * C), jnp.float32),
                   pltpu.SemaphoreType.DMA,
                   pltpu.SemaphoreType.DMA],
    compiler_params=_sc_params,
)
def _sc2_scatter(ii2, e1, e3, z16, z48, o1, o3,
                 iv, b1, b3, acc1, acc3, s1, s2):
    cid = lax.axis_index("c")
    sid = lax.axis_index("s")
    wid = sid * 2 + cid

    @pl.when(sid == 0)
    def _():
        pltpu.sync_copy(z16, acc1)
        pltpu.sync_copy(z48, acc3)

    plsc.subcore_barrier()

    def step(k, carry):
        c = wid + NWORKERS * k

        @pl.when(c < NCHUNKS)
        def _():
            pltpu.sync_copy(ii2.at[c], iv)
            d1 = pltpu.async_copy(e1.at[pl.ds(c * CHUNK, CHUNK)], b1, s1)
            d2 = pltpu.async_copy(e3.at[pl.ds(c * CHUNK, CHUNK)], b3, s2)
            d1.wait()
            d2.wait()
            pltpu.sync_copy(b1, acc1.at[iv], add=True)
            pltpu.sync_copy(b3, acc3.at[iv], add=True)

        return carry

    lax.fori_loop(0, STEPS, step, 0)
    plsc.subcore_barrier()

    @pl.when(sid == 0)
    def _():
        pltpu.sync_copy(acc1, o1.at[cid])
        pltpu.sync_copy(acc3, o3.at[cid])


def _tc3_body(p1n, p3n, a1, a3, T3IK, np1, np3):
    h = p1n.shape[0]
    p1a = p1n[...] + a1[:h] + a1[h:]
    p3a = p3n[...] + a3[:h] + a3[h:]
    np1[...] = p1a + p1a * p1a
    p1a3 = jnp.dot(p1a, T3IK[...], preferred_element_type=jnp.float32)
    np3[...] = p3a * p1a3


def _full_spec(shape):
    return pl.BlockSpec(shape, lambda *_: tuple(0 for _ in shape))


def kernel(p1, p3, p5, idx_i, idx_j, diff, basis,
           pp1_W, pp1_b, pi1_W1, pi1_b1, pi1_W2, ii1_W, ii1_b,
           pp3_W, pp3_b, ii3_W, ii3_b, pp5_W, pp5_b):
    f32 = jnp.float32
    del p5, pp5_W, pp5_b

    I8 = jnp.eye(8, dtype=f32)
    K8 = lambda M: jnp.kron(I8, M)
    tile3 = lambda M: jnp.tile(M, (1, 3))
    W1a, W1b = pi1_W1[:C], pi1_W1[C:]
    c_h = pp1_b @ W1a + pp1_b @ W1b + pi1_b1
    WA = pp1_W @ W1a
    WB = pp1_W @ W1b
    T = jnp.kron(jnp.ones((1, C), f32), jnp.eye(NB, dtype=f32))
    Rs = jnp.kron(jnp.eye(C, dtype=f32), jnp.ones((NB, 1), f32))
    W2K = K8(pi1_W2)
    TK = K8(T)
    RsK = K8(Rs)
    O1K = K8(ii1_W[:, :C])
    T2K = K8(tile3(ii1_W[:, C:2 * C]))
    T3K = K8(tile3(ii1_W[:, 2 * C:3 * C]))
    WbdK = jnp.kron(jnp.eye(24, dtype=f32), ii3_W)
    Md = jnp.kron(jnp.eye(24, dtype=f32), jnp.ones((1, C), f32))
    b1r = jnp.tile(ii1_b[:C], 8)[None]
    b2r = jnp.tile(jnp.tile(ii1_b[C:2 * C], 3), 8)[None]
    b3r = jnp.tile(jnp.tile(ii1_b[2 * C:3 * C], 3), 8)[None]
    bb3r = jnp.tile(jnp.tile(ii3_b, 3), 8)[None]
    T3IK = K8(tile3(jnp.eye(C, dtype=f32)))
    Wp1K = K8(pp1_W)
    WAK = K8(WA)
    WBK = K8(WB)
    Wp3K = K8(pp3_W)
    bp1r = jnp.tile(pp1_b, 8)[None]
    cAr = jnp.tile(c_h, 8)[None]
    bp3r = jnp.tile(pp3_b, 8)[None]

    p1_p = p1.reshape(NP8, 8 * C)
    p3r_p = p3.reshape(3 * N // 8, 8 * C)
    ii2 = idx_i.reshape(NCHUNKS, CHUNK)
    ij2 = idx_j.reshape(NCHUNKS, CHUNK)

    p1n_p, A_p, B_p, p3n_p = pl.pallas_call(
        _tc1_body,
        out_shape=(jax.ShapeDtypeStruct((NP8, 8 * C), f32),
                   jax.ShapeDtypeStruct((NP8, 8 * C), f32),
                   jax.ShapeDtypeStruct((NP8, 8 * C), f32),
                   jax.ShapeDtypeStruct((3 * N // 8, 8 * C), f32)),
    )(p1_p, p3r_p, Wp1K, bp1r, WAK, cAr, WBK, Wp3K, bp3r)

    An = A_p.reshape(N, C)
    Bn = B_p.reshape(N, C)
    p3n = p3n_p.reshape(N, 3 * C)
    gA, gB, g3 = _sc1_gather(An, Bn, p3n, ii2, ij2)

    TB = 400
    grid = (EP8 // TB,)
    bspec = lambda w: pl.BlockSpec((TB, w), lambda i: (i, 0))
    o1p, o3p = pl.pallas_call(
        _tc2_body,
        grid=grid,
        in_specs=[bspec(8 * C), bspec(8 * C), bspec(8 * NB), bspec(24 * C),
                  bspec(24),
                  _full_spec((8 * C, 8 * C * NB)),
                  _full_spec((8 * NB, 8 * C * NB)),
                  _full_spec((8 * C * NB, 8 * C)),
                  _full_spec((8 * C, 8 * C)),
                  _full_spec((8 * C, 24 * C)), _full_spec((8 * C, 24 * C)),
                  _full_spec((24 * C, 24 * C)), _full_spec((24, 24 * C)),
                  _full_spec((1, 8 * C)), _full_spec((1, 24 * C)),
                  _full_spec((1, 24 * C)), _full_spec((1, 24 * C))],
        out_specs=[bspec(8 * C), bspec(24 * C)],
        out_shape=(jax.ShapeDtypeStruct((EP8, 8 * C), f32),
                   jax.ShapeDtypeStruct((EP8, 24 * C), f32)),
    )(gA.reshape(EP8, 8 * C), gB.reshape(EP8, 8 * C),
      basis.reshape(EP8, 8 * NB), g3.reshape(EP8, 24 * C),
      diff.reshape(EP8, 24),
      W2K, TK, RsK, O1K, T2K, T3K, WbdK, Md, b1r, b2r, b3r, bb3r)

    z16 = jnp.zeros((N, C), f32)
    z48 = jnp.zeros((N, 3 * C), f32)
    a1, a3 = _sc2_scatter(ii2, o1p.reshape(E, C), o3p.reshape(E, 3 * C),
                          z16, z48)

    np1_p, np3_p = pl.pallas_call(
        _tc3_body,
        out_shape=(jax.ShapeDtypeStruct((NP8, 8 * C), f32),
                   jax.ShapeDtypeStruct((NP8, 24 * C), f32)),
    )(p1n_p, p3n_p.reshape(NP8, 24 * C), a1.reshape(2 * NP8, 8 * C),
      a3.reshape(2 * NP8, 24 * C), T3IK)
    return (np1_p.reshape(N, C), np3_p.reshape(N, 3, C))
